# Initial kernel scaffold; baseline (speedup 1.0000x reference)
import jax
import jax.numpy as jnp
from jax.experimental import pallas as pl


def kernel(pred, target):
    raise NotImplementedError("write your pallas kernel here")


# speedup vs baseline: 2.4991x; 2.4991x over previous
import functools

import jax
import jax.numpy as jnp
from jax.experimental import pallas as pl

_CLS = 100000
_B = 1024
_SMOOTH = 0.1
_CONF = 1.0 - _SMOOTH
_EPS = _SMOOTH / (_CLS - 2)
_BC = 2048
_NB = (_CLS + _BC - 1) // _BC


def _stats_body(pred_ref, tgt_ref, m_ref, s_ref, sp_ref, p0_ref, av_ref,
                ai_ref, tv_ref):
    j = pl.program_id(0)

    @pl.when(j == 0)
    def _init():
        neg = jnp.full((_B, 1), -jnp.inf, jnp.float32)
        zero = jnp.zeros((_B, 1), jnp.float32)
        m_ref[...] = neg
        av_ref[...] = neg
        s_ref[...] = zero
        sp_ref[...] = zero
        tv_ref[...] = zero
        ai_ref[...] = jnp.zeros((_B, 1), jnp.int32)

    x = pred_ref[...]
    gcol = jax.lax.broadcasted_iota(jnp.int32, (_B, _BC), 1) + j * _BC
    valid = gcol < _CLS
    xm = jnp.where(valid, x, -jnp.inf)

    bmax = jnp.max(xm, axis=1, keepdims=True)
    m_old = m_ref[...]
    m_new = jnp.maximum(m_old, bmax)
    s_ref[...] = (s_ref[...] * jnp.exp(m_old - m_new)
                  + jnp.sum(jnp.exp(xm - m_new), axis=1, keepdims=True))
    m_ref[...] = m_new

    sp_ref[...] += jnp.sum(jnp.where(valid, x, 0.0), axis=1, keepdims=True)

    bidx = jnp.min(jnp.where(xm == bmax, gcol, jnp.int32(2**30)),
                   axis=1, keepdims=True)
    better = bmax > av_ref[...]
    av_ref[...] = jnp.where(better, bmax, av_ref[...])
    ai_ref[...] = jnp.where(better, bidx, ai_ref[...])

    t = tgt_ref[...]
    tv_ref[...] += jnp.sum(jnp.where(gcol == t, x, 0.0),
                           axis=1, keepdims=True)

    @pl.when(j == 0)
    def _p0():
        p0_ref[...] = x[:, 0:1]


def _epi_body(m_ref, s_ref, sp_ref, p0_ref, ai_ref, tv_ref, t_ref, o_ref):
    lse = m_ref[...] + jnp.log(s_ref[...])
    tv = tv_ref[...]
    t = t_ref[...]
    ce_row = lse - _CONF * tv - _EPS * (sp_ref[...] - p0_ref[...] - tv)
    ce_row = jnp.where(t == 0, 0.0, ce_row)
    ce = jnp.sum(ce_row) * (1.0 / _B)
    d = 1.0 - jnp.exp(-ce)
    f_loss = d * d * ce

    ai = ai_ref[...]
    cp = jnp.where(ai < 5000, ai % 100, -1)
    ct = jnp.where(t < 5000, t % 100, -1)
    pen = jnp.where(ai == t, 0.0, jnp.where(cp == ct, 0.5, 1.0))
    cc = jnp.sum(pen) * (1.0 / _B)
    o_ref[...] = jnp.broadcast_to(f_loss + cc, (1, 1))


@functools.partial(jax.jit)
def kernel(pred, target):
    t2 = target.astype(jnp.int32).reshape(_B, 1)
    col = pl.BlockSpec((_B, 1), lambda j: (0, 0))
    stats = pl.pallas_call(
        _stats_body,
        grid=(_NB,),
        in_specs=[pl.BlockSpec((_B, _BC), lambda j: (0, j)), col],
        out_specs=[col] * 7,
        out_shape=[jax.ShapeDtypeStruct((_B, 1), jnp.float32)] * 5
        + [jax.ShapeDtypeStruct((_B, 1), jnp.int32),
           jax.ShapeDtypeStruct((_B, 1), jnp.float32)],
    )(pred, t2)
    m, s, sp, p0, _av, ai, tv = stats
    out = pl.pallas_call(
        _epi_body,
        out_shape=jax.ShapeDtypeStruct((1, 1), jnp.float32),
    )(m, s, sp, p0, ai, tv, t2)
    return out.reshape(())
